# baseline (speedup 1.0000x reference)
import jax
import jax.numpy as jnp
from jax import lax
from jax.experimental import pallas as pl
from jax.experimental.pallas import tpu as pltpu
from jax.experimental.pallas import tpu_sc as plsc

P_DROP = 0.1
N_FRAMES = 512
TOTAL = 512 * 3 * 224 * 224
NC, NS = 2, 16
NW = NC * NS
SHARD = TOTAL // NW
BLK = 4096
CHUNK = 7 * BLK
NCHUNK = SHARD // CHUNK
NB = 4
NSTEP = NCHUNK // NB


def _body(img_hbm, rand_hbm, out_hbm, rv,
          b0, b1, b2, b3, g0, g1, g2, g3, s0, s1, s2, s3):
    bufs = (b0, b1, b2, b3)
    gsems = (g0, g1, g2, g3)
    ssems = (s0, s1, s2, s3)

    wid = lax.axis_index("s") * NC + lax.axis_index("c")
    w0 = wid * SHARD

    pltpu.sync_copy(rand_hbm, rv)
    mregs = []
    for g in range(32):
        v = rv[pl.ds(g * 16, 16)]
        mregs.append(jnp.where(v >= P_DROP, jnp.float32(1.0),
                               jnp.float32(0.0)))

    def _vpass(buf):
        def _block(blk, carry):
            base = blk * BLK
            for tn in range(4):
                for lg in range(8):
                    m = mregs[tn * 8 + lg]
                    for sub in range(8):
                        off = base + tn * 1024 + sub * 128 + lg * 16
                        buf[pl.ds(off, 16)] = buf[pl.ds(off, 16)] * m
            return carry
        lax.fori_loop(0, CHUNK // BLK, _block, 0)

    def _chunk_slice(step, b):
        return pl.ds(w0 + (step * NB + b) * CHUNK, CHUNK)

    for b in range(NB):
        pltpu.async_copy(img_hbm.at[_chunk_slice(0, b)], bufs[b], gsems[b])

    def _step(step, carry):
        for b in range(NB):
            pltpu.make_async_copy(
                img_hbm.at[_chunk_slice(step, b)], bufs[b], gsems[b]).wait()
            _vpass(bufs[b])
            pltpu.async_copy(bufs[b], out_hbm.at[_chunk_slice(step, b)],
                             ssems[b])
        for b in range(NB):
            pltpu.make_async_copy(
                bufs[b], out_hbm.at[_chunk_slice(step, b)], ssems[b]).wait()

            @pl.when(step + 1 < NSTEP)
            def _next():
                pltpu.async_copy(img_hbm.at[_chunk_slice(step + 1, b)],
                                 bufs[b], gsems[b])
        return carry

    lax.fori_loop(0, NSTEP, _step, 0)


def _drop_frames_sc(img_flat, rand_nums):
    mesh = plsc.VectorSubcoreMesh(core_axis_name="c", subcore_axis_name="s")
    run = pl.kernel(
        _body,
        mesh=mesh,
        out_type=jax.ShapeDtypeStruct((TOTAL,), jnp.float32),
        scratch_types=[
            pltpu.VMEM((N_FRAMES,), jnp.float32),
        ] + [pltpu.VMEM((CHUNK,), jnp.float32)] * NB
          + [pltpu.SemaphoreType.DMA] * (2 * NB),
    )
    return run(img_flat, rand_nums)


def kernel(img, rand_nums):
    x = jnp.transpose(img, (1, 2, 3, 0))
    x = x.reshape(3, 224, 28, 8, 4, 128)
    x = jnp.transpose(x, (0, 1, 2, 4, 3, 5))
    flat = x.reshape(-1)
    o = _drop_frames_sc(flat, rand_nums)
    y = o.reshape(3, 224, 28, 4, 8, 128)
    y = jnp.transpose(y, (0, 1, 2, 4, 3, 5))
    y = y.reshape(3, 224, 224, 512)
    return jnp.transpose(y, (3, 0, 1, 2))
